# Initial kernel scaffold; baseline (speedup 1.0000x reference)
import jax
import jax.numpy as jnp
from jax.experimental import pallas as pl


def kernel(x, edge_index, W_l, W_r, att, bias, ln_weight, ln_bias):
    raise NotImplementedError("write your pallas kernel here")


# speedup vs baseline: 6.9579x; 6.9579x over previous
import jax
import jax.numpy as jnp
from jax import lax
from jax.experimental import pallas as pl
from jax.experimental.pallas import tpu as pltpu
from jax.experimental.pallas import tpu_sc as plsc

_N = 10000
_E = 320000
_C = 128
_NC = 2
_NS = 16
_NW = _NC * _NS
_L = 16
_EPT = _E // _NW
_K = 80
_NCHUNK = _EPT // _K
_NPAD = 10240
_RPT = _NPAD // _NS
_NEG = -1e30


def _proj_body(x_ref, wl_ref, wr_ref, xl_ref, xr_ref):
    x = x_ref[...]
    xl_ref[...] = jnp.dot(x, wl_ref[...], preferred_element_type=jnp.float32)
    xr_ref[...] = jnp.dot(x, wr_ref[...], preferred_element_type=jnp.float32)


def _proj(x, W_l, W_r):
    return pl.pallas_call(
        _proj_body,
        out_shape=[
            jax.ShapeDtypeStruct((_N, _C), jnp.float32),
            jax.ShapeDtypeStruct((_N, _C), jnp.float32),
        ],
    )(x, W_l, W_r)


def _sc_body(xl_hbm, xr_hbm, att_hbm, srcf_hbm, dstf_hbm,
             m_out, d_out, s_out, l_hbm, m_stage,
             m_loc, d_loc, rl, rr, mro, mbuf0, mbuf1, lbuf, src_c, dst_c,
             tbuf, att_vm, s_sh, sem):
    cid = lax.axis_index("c")
    sid = lax.axis_index("s")
    wid = cid * _NS + sid
    ebase = wid * _EPT

    pltpu.sync_copy(att_hbm, att_vm)
    att_s = [att_vm[pl.ds(f * _L, _L)] for f in range(_C // _L)]

    def _init(i, _):
        m_loc[pl.ds(i * _L, _L)] = jnp.full((_L,), _NEG, jnp.float32)
        d_loc[pl.ds(i * _L, _L)] = jnp.zeros((_L,), jnp.float32)
        return 0
    lax.fori_loop(0, _NPAD // _L, _init, 0)

    iota_row = lax.iota(jnp.int32, _L) * _L

    def _chunk_a(c, _):
        pltpu.sync_copy(srcf_hbm.at[pl.ds(ebase + c * _K, _K)], src_c)
        pltpu.sync_copy(dstf_hbm.at[pl.ds(ebase + c * _K, _K)], dst_c)
        pltpu.async_copy(xl_hbm.at[src_c], rl, sem).wait()
        pltpu.async_copy(xr_hbm.at[dst_c], rr, sem).wait()

        def _group(g, _):
            e0 = g * _L
            for i in range(_L):
                acc = None
                for f in range(_C // _L):
                    v = rl[e0 + i, pl.ds(f * _L, _L)] \
                        + rr[e0 + i, pl.ds(f * _L, _L)]
                    lr = 0.6 * v + 0.4 * jnp.abs(v)
                    t = lr * att_s[f]
                    acc = t if acc is None else acc + t
                tbuf[pl.ds(i * _L, _L)] = acc
            tot = None
            for j in range(_L):
                col = plsc.load_gather(tbuf, [iota_row + j])
                tot = col if tot is None else tot + col
            lbuf[pl.ds(e0, _L)] = tot
            d16 = dst_c[pl.ds(e0, _L)]
            cur = plsc.load_gather(m_loc, [d16])
            plsc.store_scatter(m_loc, [d16], jnp.maximum(cur, tot))
            return 0
        lax.fori_loop(0, _K // _L, _group, 0)
        pltpu.sync_copy(lbuf, l_hbm.at[pl.ds(ebase + c * _K, _K)])
        return 0
    lax.fori_loop(0, _NCHUNK, _chunk_a, 0)

    pltpu.sync_copy(m_loc, m_stage.at[pl.ds(wid * _NPAD, _NPAD)])
    plsc.subcore_barrier()
    rbase = sid * _RPT
    bufs = [mbuf0, mbuf1]
    sbase = cid * _NS * _NPAD + rbase
    cps = [pltpu.async_copy(m_stage.at[pl.ds(sbase + t * _NPAD, _RPT)],
                            bufs[t % 2], sem) for t in (0, 1)]
    for t in range(_NS):
        cps[t % 2].wait()
        if t + 2 < _NS:
            cps[t % 2] = pltpu.async_copy(
                m_stage.at[pl.ds(sbase + (t + 2) * _NPAD, _RPT)],
                bufs[t % 2], sem)

        def _red(i, _, _t=t):
            v = bufs[_t % 2][pl.ds(i * _L, _L)]
            if _t == 0:
                mro[pl.ds(i * _L, _L)] = v
            else:
                mro[pl.ds(i * _L, _L)] = jnp.maximum(mro[pl.ds(i * _L, _L)], v)
            return 0
        lax.fori_loop(0, _RPT // _L, _red, 0)
    pltpu.sync_copy(mro, m_out.at[pl.ds(cid * _NPAD + rbase, _RPT)])
    plsc.subcore_barrier()
    pltpu.sync_copy(m_out.at[pl.ds(cid * _NPAD, _NPAD)], m_loc)

    def _z(i, _):
        for f in range(_C // _L):
            rl[i, pl.ds(f * _L, _L)] = jnp.zeros((_L,), jnp.float32)
        return 0
    lax.fori_loop(0, _K, _z, 0)

    def _z2(k, _):
        pltpu.sync_copy(rl, s_sh.at[pl.ds(rbase + k * _K, _K)])
        return 0
    lax.fori_loop(0, _RPT // _K, _z2, 0)
    plsc.subcore_barrier()

    def _pc(c, _):
        pltpu.sync_copy(srcf_hbm.at[pl.ds(ebase + c * _K, _K)], src_c)
        pltpu.sync_copy(dstf_hbm.at[pl.ds(ebase + c * _K, _K)], dst_c)
        pltpu.sync_copy(l_hbm.at[pl.ds(ebase + c * _K, _K)], lbuf)
        pltpu.async_copy(xl_hbm.at[src_c], rl, sem).wait()

        def _group(g, _):
            e0 = g * _L
            d16 = dst_c[pl.ds(e0, _L)]
            l16 = lbuf[pl.ds(e0, _L)]
            m16 = plsc.load_gather(m_loc, [d16])
            u16 = jnp.exp(l16 - m16)
            plsc.addupdate_scatter(d_loc, [d16], u16)
            for i in range(_L):
                u = u16[i]
                for f in range(_C // _L):
                    rl[e0 + i, pl.ds(f * _L, _L)] = \
                        rl[e0 + i, pl.ds(f * _L, _L)] * u
            return 0
        lax.fori_loop(0, _K // _L, _group, 0)
        pltpu.sync_copy(rl, s_sh.at[dst_c], add=True)
        return 0
    lax.fori_loop(0, _NCHUNK, _pc, 0)
    pltpu.sync_copy(d_loc, d_out.at[pl.ds(wid * _NPAD, _NPAD)])
    plsc.subcore_barrier()
    pltpu.sync_copy(s_sh.at[pl.ds(rbase, _RPT)],
                    s_out.at[pl.ds(cid * _NPAD + rbase, _RPT)])


def _sc_call(xl, xr, att_v, src_f, dst_f):
    outs = pl.kernel(
        _sc_body,
        out_type=[
            jax.ShapeDtypeStruct((_NC * _NPAD,), jnp.float32),
            jax.ShapeDtypeStruct((_NC * _NS * _NPAD,), jnp.float32),
            jax.ShapeDtypeStruct((_NC * _NPAD, _C), jnp.float32),
            jax.ShapeDtypeStruct((_E,), jnp.float32),
            jax.ShapeDtypeStruct((_NC * _NS * _NPAD,), jnp.float32),
        ],
        mesh=plsc.VectorSubcoreMesh(core_axis_name="c", subcore_axis_name="s"),
        compiler_params=pltpu.CompilerParams(needs_layout_passes=False),
        scratch_types=[
            pltpu.VMEM((_NPAD,), jnp.float32),
            pltpu.VMEM((_NPAD,), jnp.float32),
            pltpu.VMEM((_K, _C), jnp.float32),
            pltpu.VMEM((_K, _C), jnp.float32),
            pltpu.VMEM((_RPT,), jnp.float32),
            pltpu.VMEM((_RPT,), jnp.float32),
            pltpu.VMEM((_RPT,), jnp.float32),
            pltpu.VMEM((_K,), jnp.float32),
            pltpu.VMEM((_K,), jnp.int32),
            pltpu.VMEM((_K,), jnp.int32),
            pltpu.VMEM((_L * _L,), jnp.float32),
            pltpu.VMEM((_C,), jnp.float32),
            pltpu.VMEM_SHARED((_NPAD, _C), jnp.float32),
            pltpu.SemaphoreType.DMA,
        ],
    )(xl, xr, att_v, src_f, dst_f)
    return (outs[0].reshape(_NC, _NPAD),
            outs[1].reshape(_NC, _NS, _NPAD),
            outs[2].reshape(_NC, _NPAD, _C))


def _merge_body(m_ref, d_ref, s_ref, bias_ref, lnw_ref, lnb_ref, out_ref):
    m = m_ref[...]
    mm = jnp.max(m, axis=0, keepdims=True)
    w = jnp.exp(m - mm)
    dsum = jnp.sum(d_ref[...], axis=1)
    den = jnp.sum(dsum * w, axis=0)
    s = jnp.sum(s_ref[...] * w[:, :, None], axis=0)
    pre = s / (den[:, None] + 1e-16) + bias_ref[...][None, :]
    pre = pre[:_N]
    mu = jnp.mean(pre)
    xc = pre - mu
    var = jnp.mean(xc * xc)
    out_ref[...] = xc * lax.rsqrt(var + 1e-5) * lnw_ref[...][None, :] \
        + lnb_ref[...][None, :]


def _merge(m_p, d_p, s_p, bias, ln_weight, ln_bias):
    return pl.pallas_call(
        _merge_body,
        out_shape=jax.ShapeDtypeStruct((_N, _C), jnp.float32),
    )(m_p, d_p, s_p, bias, ln_weight, ln_bias)


def kernel(x, edge_index, W_l, W_r, att, bias, ln_weight, ln_bias):
    xl, xr = _proj(x, W_l, W_r)
    att_v = att.reshape(_C)
    m_p, d_p, s_p = _sc_call(xl, xr, att_v, edge_index[0], edge_index[1])
    return _merge(m_p, d_p, s_p, bias, ln_weight, ln_bias)
